# baseline (speedup 1.0000x reference)
import functools

import jax
import jax.numpy as jnp
from jax import lax
from jax.experimental import pallas as pl
from jax.experimental.pallas import tpu as pltpu
from jax.experimental.pallas import tpu_sc as plsc

N = 10000
E = 320000
D = 128
DOUT = 64
NC = 2
NS = 16
NW = NC * NS
CHUNK = 128
NCHUNK = 80
EPAD = NW * NCHUNK * CHUNK
NPAD = 10112
RPT = NPAD // NS
BR = 2000


@functools.cache
def _mesh():
    return plsc.VectorSubcoreMesh(core_axis_name="c", subcore_axis_name="s",
                                  num_cores=NC, num_subcores=NS)


NTOT = EPAD // (NS * CHUNK)
HC2 = NTOT // 2


def _make_edge_body(half):
    def body(idx_hbm, z_hbm, zeros_hbm, ones_hbm, out_hbm,
             idx_v, rows_v, acc, sem):
        c = lax.axis_index("c")
        s = lax.axis_index("s")

        @pl.when(c == 0)
        def _():
            pltpu.sync_copy(zeros_hbm.at[pl.ds(s * RPT, RPT)],
                            acc.at[pl.ds(s * RPT, RPT)])
            pltpu.sync_copy(idx_hbm.at[s, pl.ds(half * HC2, HC2)], idx_v)
            plsc.subcore_barrier()

            def step(j, carry):
                pltpu.async_copy(z_hbm.at[idx_v.at[j, 0]], rows_v, sem).wait()
                pltpu.sync_copy(rows_v, acc.at[idx_v.at[j, 1]], add=True)
                return carry

            lax.fori_loop(0, HC2, step, 0)
            plsc.subcore_barrier()
            pltpu.sync_copy(acc.at[pl.ds(s * RPT, RPT)],
                            out_hbm.at[0, pl.ds(s * RPT, RPT)])

        if half == 0:
            @pl.when(c == 1)
            def _():
                pltpu.sync_copy(zeros_hbm.at[pl.ds(s * RPT, RPT)],
                                acc.at[pl.ds(s * RPT, RPT)])
                pltpu.sync_copy(ones_hbm, rows_v)
                plsc.subcore_barrier()
                for hh in range(2):
                    pltpu.sync_copy(idx_hbm.at[s, pl.ds(hh * HC2, HC2)],
                                    idx_v)

                    def step1(j, carry):
                        pltpu.sync_copy(rows_v, acc.at[idx_v.at[j, 1]],
                                        add=True)
                        return carry

                    lax.fori_loop(0, HC2, step1, 0)
                plsc.subcore_barrier()
                pltpu.sync_copy(acc.at[pl.ds(s * RPT, RPT)],
                                out_hbm.at[1, pl.ds(s * RPT, RPT)])

    return body


@functools.cache
def _edge_kernel(half):
    return pl.kernel(
        _make_edge_body(half),
        out_type=jax.ShapeDtypeStruct((NC, NPAD, D), jnp.float32),
        mesh=_mesh(),
        scratch_types=[
            pltpu.VMEM((HC2, 2, CHUNK), jnp.int32),
            pltpu.VMEM((CHUNK, D), jnp.float32),
            pltpu.VMEM_SHARED((NPAD, D), jnp.float32),
            pltpu.SemaphoreType.DMA,
        ],
    )


def _ln(h, g, b):
    mu = jnp.mean(h, axis=-1, keepdims=True)
    var = jnp.mean((h - mu) ** 2, axis=-1, keepdims=True)
    return (h - mu) * lax.rsqrt(var + 1e-5) * g + b


def _proj_body(x_ref, w_ref, b_ref, g_ref, bb_ref, h_ref, z_ref):
    h = jnp.dot(x_ref[...], w_ref[...],
                preferred_element_type=jnp.float32) + b_ref[...]
    h_ref[...] = h
    z_ref[...] = _ln(h, g_ref[...], bb_ref[...])


_proj_ln = pl.pallas_call(
    _proj_body,
    grid=(N // BR,),
    in_specs=[
        pl.BlockSpec((BR, D), lambda i: (i, 0)),
        pl.BlockSpec((D, D), lambda i: (0, 0)),
        pl.BlockSpec((1, D), lambda i: (0, 0)),
        pl.BlockSpec((1, D), lambda i: (0, 0)),
        pl.BlockSpec((1, D), lambda i: (0, 0)),
    ],
    out_specs=[pl.BlockSpec((BR, D), lambda i: (i, 0)),
               pl.BlockSpec((BR, D), lambda i: (i, 0))],
    out_shape=[jax.ShapeDtypeStruct((N, D), jnp.float32),
               jax.ShapeDtypeStruct((N, D), jnp.float32)],
)


def _sage_common(h_ref, z_ref, p_ref, dg_ref, wn_ref, ws_ref, bc_ref):
    deg = jnp.maximum(p_ref[1, :, 0:1], 1.0)
    agg = (p_ref[0] + dg_ref[0]) / deg
    conv = (jnp.dot(agg, wn_ref[...], preferred_element_type=jnp.float32)
            + jnp.dot(z_ref[...], ws_ref[...], preferred_element_type=jnp.float32)
            + bc_ref[...])
    return jnp.maximum(h_ref[...] + conv, 0.0)


def _mid_body(h_ref, z_ref, p_ref, dg_ref, wn_ref, ws_ref, bc_ref,
              g_ref, bb_ref, ho_ref, zo_ref):
    hn = _sage_common(h_ref, z_ref, p_ref, dg_ref, wn_ref, ws_ref, bc_ref)
    ho_ref[...] = hn
    zo_ref[...] = _ln(hn, g_ref[...], bb_ref[...])


_SAGE_SPECS = [
    pl.BlockSpec((BR, D), lambda i: (i, 0)),
    pl.BlockSpec((BR, D), lambda i: (i, 0)),
    pl.BlockSpec((NC, BR, D), lambda i: (0, i, 0)),
    pl.BlockSpec((1, BR, D), lambda i: (0, i, 0)),
    pl.BlockSpec((D, D), lambda i: (0, 0)),
    pl.BlockSpec((D, D), lambda i: (0, 0)),
    pl.BlockSpec((1, D), lambda i: (0, 0)),
]

_mid_layer = pl.pallas_call(
    _mid_body,
    grid=(N // BR,),
    in_specs=_SAGE_SPECS + [
        pl.BlockSpec((1, D), lambda i: (0, 0)),
        pl.BlockSpec((1, D), lambda i: (0, 0)),
    ],
    out_specs=[pl.BlockSpec((BR, D), lambda i: (i, 0)),
               pl.BlockSpec((BR, D), lambda i: (i, 0))],
    out_shape=[jax.ShapeDtypeStruct((N, D), jnp.float32),
               jax.ShapeDtypeStruct((N, D), jnp.float32)],
)


def _last_body(h_ref, z_ref, p_ref, dg_ref, wn_ref, ws_ref, bc_ref,
               wo_ref, bo_ref, o_ref):
    hn = _sage_common(h_ref, z_ref, p_ref, dg_ref, wn_ref, ws_ref, bc_ref)
    o_ref[...] = jnp.dot(hn, wo_ref[...],
                         preferred_element_type=jnp.float32) + bo_ref[...]


_last_layer = pl.pallas_call(
    _last_body,
    grid=(N // BR,),
    in_specs=_SAGE_SPECS + [
        pl.BlockSpec((D, DOUT), lambda i: (0, 0)),
        pl.BlockSpec((1, DOUT), lambda i: (0, 0)),
    ],
    out_specs=pl.BlockSpec((BR, DOUT), lambda i: (i, 0)),
    out_shape=jax.ShapeDtypeStruct((N, DOUT), jnp.float32),
)


def kernel(x, edge_index, W_in, b_in, ln_g, ln_b, W_neigh, W_self, b_conv,
           W_out, b_out):
    i32 = jnp.int32
    src = edge_index[0].astype(i32)
    dst = edge_index[1].astype(i32)
    src_p = jnp.concatenate([src, jnp.zeros((EPAD - E,), i32)])
    dst_p = jnp.concatenate([dst, jnp.full((EPAD - E,), N, i32)])
    idx_p = jnp.stack([src_p.reshape(NS, NTOT, CHUNK),
                       dst_p.reshape(NS, NTOT, CHUNK)], axis=2)

    zeros_d = jnp.zeros((NPAD, D), jnp.float32)
    ones_d = jnp.ones((CHUNK, D), jnp.float32)

    h, z = _proj_ln(x, W_in, b_in.reshape(1, D),
                    ln_g[0].reshape(1, D), ln_b[0].reshape(1, D))

    out = None
    for i in range(W_self.shape[0]):
        parts = _edge_kernel(0)(idx_p, z, zeros_d, ones_d)
        degp = _edge_kernel(1)(idx_p, z, zeros_d, ones_d)
        if i + 1 < W_self.shape[0]:
            h, z = _mid_layer(h, z, parts, degp, W_neigh[i], W_self[i],
                              b_conv[i].reshape(1, D),
                              ln_g[i + 1].reshape(1, D),
                              ln_b[i + 1].reshape(1, D))
        else:
            out = _last_layer(h, z, parts, degp, W_neigh[i], W_self[i],
                              b_conv[i].reshape(1, D), W_out,
                              b_out.reshape(1, DOUT))
    return out


# speedup vs baseline: 1.8726x; 1.8726x over previous
import functools

import jax
import jax.numpy as jnp
from jax import lax
from jax.experimental import pallas as pl
from jax.experimental.pallas import tpu as pltpu
from jax.experimental.pallas import tpu_sc as plsc

N = 10000
E = 320000
D = 128
DOUT = 64
NC = 2
NS = 16
NW = NC * NS
CHUNK = 128
NCHUNK = 80
EPAD = NW * NCHUNK * CHUNK
NPAD = 10112
RPT = NPAD // NS
BR = 2000


@functools.cache
def _mesh():
    return plsc.VectorSubcoreMesh(core_axis_name="c", subcore_axis_name="s",
                                  num_cores=NC, num_subcores=NS)


NTOT = EPAD // (NS * CHUNK)
HC2 = NTOT // 2


def _make_edge_body(half):
    def body(idx_hbm, z_hbm, zeros_hbm, ones_hbm, out_hbm,
             idx_v, rows_v, acc, sem):
        c = lax.axis_index("c")
        s = lax.axis_index("s")

        @pl.when(c == 0)
        def _():
            pltpu.sync_copy(zeros_hbm.at[pl.ds(s * RPT, RPT)],
                            acc.at[pl.ds(s * RPT, RPT)])
            pltpu.sync_copy(idx_hbm.at[s, pl.ds(half * HC2, HC2)], idx_v)
            plsc.subcore_barrier()

            def step(j, carry):
                pltpu.async_copy(z_hbm.at[idx_v.at[j, 0]], rows_v, sem).wait()
                pltpu.sync_copy(rows_v, acc.at[idx_v.at[j, 1]], add=True)
                return carry

            lax.fori_loop(0, HC2, step, 0)
            plsc.subcore_barrier()
            pltpu.sync_copy(acc.at[pl.ds(s * RPT, RPT)],
                            out_hbm.at[0, pl.ds(s * RPT, RPT)])

        if half == 0:
            @pl.when(c == 1)
            def _():
                pltpu.sync_copy(zeros_hbm.at[pl.ds(s * RPT, RPT)],
                                acc.at[pl.ds(s * RPT, RPT)])
                pltpu.sync_copy(ones_hbm, rows_v)
                plsc.subcore_barrier()
                for hh in range(2):
                    pltpu.sync_copy(idx_hbm.at[s, pl.ds(hh * HC2, HC2)],
                                    idx_v)

                    def step1(j, carry):
                        pltpu.sync_copy(rows_v, acc.at[idx_v.at[j, 1]],
                                        add=True)
                        return carry

                    lax.fori_loop(0, HC2, step1, 0)
                plsc.subcore_barrier()
                pltpu.sync_copy(acc.at[pl.ds(s * RPT, RPT)],
                                out_hbm.at[1, pl.ds(s * RPT, RPT)])

    return body


@functools.cache
def _edge_kernel(half):
    return pl.kernel(
        _make_edge_body(half),
        out_type=jax.ShapeDtypeStruct((NC, NPAD, D), jnp.float32),
        mesh=_mesh(),
        scratch_types=[
            pltpu.VMEM((HC2, 2, CHUNK), jnp.int32),
            pltpu.VMEM((CHUNK, D), jnp.float32),
            pltpu.VMEM_SHARED((NPAD, D), jnp.float32),
            pltpu.SemaphoreType.DMA,
        ],
    )


def _ln(h, g, b):
    mu = jnp.mean(h, axis=-1, keepdims=True)
    var = jnp.mean((h - mu) ** 2, axis=-1, keepdims=True)
    return (h - mu) * lax.rsqrt(var + 1e-5) * g + b


def _proj_body(x_ref, w_ref, b_ref, g_ref, bb_ref, h_ref, z_ref):
    h = jnp.dot(x_ref[...], w_ref[...],
                preferred_element_type=jnp.float32) + b_ref[...]
    h_ref[...] = h
    z_ref[...] = _ln(h, g_ref[...], bb_ref[...])


_proj_ln = pl.pallas_call(
    _proj_body,
    grid=(N // BR,),
    in_specs=[
        pl.BlockSpec((BR, D), lambda i: (i, 0)),
        pl.BlockSpec((D, D), lambda i: (0, 0)),
        pl.BlockSpec((1, D), lambda i: (0, 0)),
        pl.BlockSpec((1, D), lambda i: (0, 0)),
        pl.BlockSpec((1, D), lambda i: (0, 0)),
    ],
    out_specs=[pl.BlockSpec((BR, D), lambda i: (i, 0)),
               pl.BlockSpec((BR, D), lambda i: (i, 0))],
    out_shape=[jax.ShapeDtypeStruct((N, D), jnp.float32),
               jax.ShapeDtypeStruct((N, D), jnp.float32)],
)


def _sage_common(h_ref, z_ref, p_ref, dg_ref, wn_ref, ws_ref, bc_ref):
    deg = jnp.maximum(p_ref[1, :, 0:1], 1.0)
    agg = (p_ref[0] + dg_ref[0]) / deg
    conv = (jnp.dot(agg, wn_ref[...], preferred_element_type=jnp.float32)
            + jnp.dot(z_ref[...], ws_ref[...], preferred_element_type=jnp.float32)
            + bc_ref[...])
    return jnp.maximum(h_ref[...] + conv, 0.0)


def _mid_body(h_ref, z_ref, p_ref, dg_ref, wn_ref, ws_ref, bc_ref,
              g_ref, bb_ref, ho_ref, zo_ref):
    hn = _sage_common(h_ref, z_ref, p_ref, dg_ref, wn_ref, ws_ref, bc_ref)
    ho_ref[...] = hn
    zo_ref[...] = _ln(hn, g_ref[...], bb_ref[...])


_SAGE_SPECS = [
    pl.BlockSpec((BR, D), lambda i: (i, 0)),
    pl.BlockSpec((BR, D), lambda i: (i, 0)),
    pl.BlockSpec((NC, BR, D), lambda i: (0, i, 0)),
    pl.BlockSpec((1, BR, D), lambda i: (0, i, 0)),
    pl.BlockSpec((D, D), lambda i: (0, 0)),
    pl.BlockSpec((D, D), lambda i: (0, 0)),
    pl.BlockSpec((1, D), lambda i: (0, 0)),
]

_mid_layer = pl.pallas_call(
    _mid_body,
    grid=(N // BR,),
    in_specs=_SAGE_SPECS + [
        pl.BlockSpec((1, D), lambda i: (0, 0)),
        pl.BlockSpec((1, D), lambda i: (0, 0)),
    ],
    out_specs=[pl.BlockSpec((BR, D), lambda i: (i, 0)),
               pl.BlockSpec((BR, D), lambda i: (i, 0))],
    out_shape=[jax.ShapeDtypeStruct((N, D), jnp.float32),
               jax.ShapeDtypeStruct((N, D), jnp.float32)],
)


def _last_body(h_ref, z_ref, p_ref, dg_ref, wn_ref, ws_ref, bc_ref,
               wo_ref, bo_ref, o_ref):
    hn = _sage_common(h_ref, z_ref, p_ref, dg_ref, wn_ref, ws_ref, bc_ref)
    o_ref[...] = jnp.dot(hn, wo_ref[...],
                         preferred_element_type=jnp.float32) + bo_ref[...]


_last_layer = pl.pallas_call(
    _last_body,
    grid=(N // BR,),
    in_specs=_SAGE_SPECS + [
        pl.BlockSpec((D, DOUT), lambda i: (0, 0)),
        pl.BlockSpec((1, DOUT), lambda i: (0, 0)),
    ],
    out_specs=pl.BlockSpec((BR, DOUT), lambda i: (i, 0)),
    out_shape=jax.ShapeDtypeStruct((N, DOUT), jnp.float32),
)


def kernel(x, edge_index, W_in, b_in, ln_g, ln_b, W_neigh, W_self, b_conv,
           W_out, b_out):
    i32 = jnp.int32
    src = edge_index[0].astype(i32)
    dst = edge_index[1].astype(i32)
    pad = jnp.arange(EPAD - E, dtype=i32)
    src_p = jnp.concatenate([src, pad % N])
    dst_p = jnp.concatenate([dst, N + pad % (NPAD - N)])
    idx_p = jnp.stack([src_p.reshape(NS, NTOT, CHUNK),
                       dst_p.reshape(NS, NTOT, CHUNK)], axis=2)

    zeros_d = jnp.zeros((NPAD, D), jnp.float32)
    ones_d = jnp.ones((CHUNK, D), jnp.float32)

    h, z = _proj_ln(x, W_in, b_in.reshape(1, D),
                    ln_g[0].reshape(1, D), ln_b[0].reshape(1, D))

    out = None
    for i in range(W_self.shape[0]):
        parts = _edge_kernel(0)(idx_p, z, zeros_d, ones_d)
        degp = _edge_kernel(1)(idx_p, z, zeros_d, ones_d)
        if i + 1 < W_self.shape[0]:
            h, z = _mid_layer(h, z, parts, degp, W_neigh[i], W_self[i],
                              b_conv[i].reshape(1, D),
                              ln_g[i + 1].reshape(1, D),
                              ln_b[i + 1].reshape(1, D))
        else:
            out = _last_layer(h, z, parts, degp, W_neigh[i], W_self[i],
                              b_conv[i].reshape(1, D), W_out,
                              b_out.reshape(1, DOUT))
    return out


# speedup vs baseline: 3.0678x; 1.6383x over previous
import functools

import jax
import jax.numpy as jnp
from jax import lax
from jax.experimental import pallas as pl
from jax.experimental.pallas import tpu as pltpu
from jax.experimental.pallas import tpu_sc as plsc

N = 10000
E = 320000
D = 128
DOUT = 64
NC = 2
NS = 16
NW = NC * NS
CHUNK = 128
NCHUNK = 80
EPAD = NW * NCHUNK * CHUNK
NPAD = 10112
RPT = NPAD // NS
BR = 2000


@functools.cache
def _mesh():
    return plsc.VectorSubcoreMesh(core_axis_name="c", subcore_axis_name="s",
                                  num_cores=NC, num_subcores=NS)


def _edge_body(idx_hbm, z_hbm, zeros_hbm, out_hbm, idx_v, rows_v, acc, sem):
    c = lax.axis_index("c")
    s = lax.axis_index("s")
    wid = s * NC + c
    pltpu.sync_copy(zeros_hbm.at[pl.ds(s * RPT, RPT)],
                    acc.at[pl.ds(s * RPT, RPT)])
    pltpu.sync_copy(idx_hbm.at[wid], idx_v)
    plsc.subcore_barrier()

    def step(j, carry):
        pltpu.async_copy(z_hbm.at[idx_v.at[j, 0]], rows_v, sem).wait()
        pltpu.sync_copy(rows_v, acc.at[idx_v.at[j, 1]], add=True)
        return carry

    lax.fori_loop(0, NCHUNK, step, 0)
    plsc.subcore_barrier()
    pltpu.sync_copy(acc.at[pl.ds(s * RPT, RPT)],
                    out_hbm.at[c, pl.ds(s * RPT, RPT)])


@functools.cache
def _edge_kernel():
    return pl.kernel(
        _edge_body,
        out_type=jax.ShapeDtypeStruct((NC, NPAD, D), jnp.float32),
        mesh=_mesh(),
        scratch_types=[
            pltpu.VMEM((NCHUNK, 2, CHUNK), jnp.int32),
            pltpu.VMEM((CHUNK, D), jnp.float32),
            pltpu.VMEM_SHARED((NPAD, D), jnp.float32),
            pltpu.SemaphoreType.DMA,
        ],
    )


def _deg_body(dst_hbm, ones_hbm, zeros_hbm, out_hbm, dst_v, ones_v, acc):
    c = lax.axis_index("c")
    s = lax.axis_index("s")
    wid = s * NC + c
    pltpu.sync_copy(zeros_hbm.at[pl.ds(s * RPT, RPT)],
                    acc.at[pl.ds(s * RPT, RPT)])
    pltpu.sync_copy(ones_hbm, ones_v)
    pltpu.sync_copy(dst_hbm.at[wid], dst_v)
    plsc.subcore_barrier()

    def step(j, carry):
        pltpu.sync_copy(ones_v, acc.at[dst_v.at[j]], add=True)
        return carry

    lax.fori_loop(0, NCHUNK, step, 0)
    plsc.subcore_barrier()
    pltpu.sync_copy(acc.at[pl.ds(s * RPT, RPT)],
                    out_hbm.at[c, pl.ds(s * RPT, RPT)])


@functools.cache
def _deg_kernel():
    return pl.kernel(
        _deg_body,
        out_type=jax.ShapeDtypeStruct((NC, NPAD, D), jnp.float32),
        mesh=_mesh(),
        scratch_types=[
            pltpu.VMEM((NCHUNK, CHUNK), jnp.int32),
            pltpu.VMEM((CHUNK, D), jnp.float32),
            pltpu.VMEM_SHARED((NPAD, D), jnp.float32),
        ],
    )


def _ln(h, g, b):
    mu = jnp.mean(h, axis=-1, keepdims=True)
    var = jnp.mean((h - mu) ** 2, axis=-1, keepdims=True)
    return (h - mu) * lax.rsqrt(var + 1e-5) * g + b


def _proj_body(x_ref, w_ref, b_ref, g_ref, bb_ref, h_ref, z_ref):
    h = jnp.dot(x_ref[...], w_ref[...],
                preferred_element_type=jnp.float32) + b_ref[...]
    h_ref[...] = h
    z_ref[...] = _ln(h, g_ref[...], bb_ref[...])


_proj_ln = pl.pallas_call(
    _proj_body,
    grid=(N // BR,),
    in_specs=[
        pl.BlockSpec((BR, D), lambda i: (i, 0)),
        pl.BlockSpec((D, D), lambda i: (0, 0)),
        pl.BlockSpec((1, D), lambda i: (0, 0)),
        pl.BlockSpec((1, D), lambda i: (0, 0)),
        pl.BlockSpec((1, D), lambda i: (0, 0)),
    ],
    out_specs=[pl.BlockSpec((BR, D), lambda i: (i, 0)),
               pl.BlockSpec((BR, D), lambda i: (i, 0))],
    out_shape=[jax.ShapeDtypeStruct((N, D), jnp.float32),
               jax.ShapeDtypeStruct((N, D), jnp.float32)],
)


def _sage_common(h_ref, z_ref, p_ref, dg_ref, wn_ref, ws_ref, bc_ref):
    deg = jnp.maximum(dg_ref[0, :, 0:1] + dg_ref[1, :, 0:1], 1.0)
    agg = (p_ref[0] + p_ref[1]) / deg
    conv = (jnp.dot(agg, wn_ref[...], preferred_element_type=jnp.float32)
            + jnp.dot(z_ref[...], ws_ref[...], preferred_element_type=jnp.float32)
            + bc_ref[...])
    return jnp.maximum(h_ref[...] + conv, 0.0)


def _mid_body(h_ref, z_ref, p_ref, dg_ref, wn_ref, ws_ref, bc_ref,
              g_ref, bb_ref, ho_ref, zo_ref):
    hn = _sage_common(h_ref, z_ref, p_ref, dg_ref, wn_ref, ws_ref, bc_ref)
    ho_ref[...] = hn
    zo_ref[...] = _ln(hn, g_ref[...], bb_ref[...])


_SAGE_SPECS = [
    pl.BlockSpec((BR, D), lambda i: (i, 0)),
    pl.BlockSpec((BR, D), lambda i: (i, 0)),
    pl.BlockSpec((NC, BR, D), lambda i: (0, i, 0)),
    pl.BlockSpec((NC, BR, D), lambda i: (0, i, 0)),
    pl.BlockSpec((D, D), lambda i: (0, 0)),
    pl.BlockSpec((D, D), lambda i: (0, 0)),
    pl.BlockSpec((1, D), lambda i: (0, 0)),
]

_mid_layer = pl.pallas_call(
    _mid_body,
    grid=(N // BR,),
    in_specs=_SAGE_SPECS + [
        pl.BlockSpec((1, D), lambda i: (0, 0)),
        pl.BlockSpec((1, D), lambda i: (0, 0)),
    ],
    out_specs=[pl.BlockSpec((BR, D), lambda i: (i, 0)),
               pl.BlockSpec((BR, D), lambda i: (i, 0))],
    out_shape=[jax.ShapeDtypeStruct((N, D), jnp.float32),
               jax.ShapeDtypeStruct((N, D), jnp.float32)],
)


def _last_body(h_ref, z_ref, p_ref, dg_ref, wn_ref, ws_ref, bc_ref,
               wo_ref, bo_ref, o_ref):
    hn = _sage_common(h_ref, z_ref, p_ref, dg_ref, wn_ref, ws_ref, bc_ref)
    o_ref[...] = jnp.dot(hn, wo_ref[...],
                         preferred_element_type=jnp.float32) + bo_ref[...]


_last_layer = pl.pallas_call(
    _last_body,
    grid=(N // BR,),
    in_specs=_SAGE_SPECS + [
        pl.BlockSpec((D, DOUT), lambda i: (0, 0)),
        pl.BlockSpec((1, DOUT), lambda i: (0, 0)),
    ],
    out_specs=pl.BlockSpec((BR, DOUT), lambda i: (i, 0)),
    out_shape=jax.ShapeDtypeStruct((N, DOUT), jnp.float32),
)


def kernel(x, edge_index, W_in, b_in, ln_g, ln_b, W_neigh, W_self, b_conv,
           W_out, b_out):
    i32 = jnp.int32
    src = edge_index[0].astype(i32)
    dst = edge_index[1].astype(i32)
    pad = jnp.arange(EPAD - E, dtype=i32)
    src_p = jnp.concatenate([src, pad % N])
    dst_p = jnp.concatenate([dst, N + pad % (NPAD - N)])
    idx_p = jnp.stack([src_p.reshape(NW, NCHUNK, CHUNK),
                       dst_p.reshape(NW, NCHUNK, CHUNK)], axis=2)
    dst_c = dst_p.reshape(NW, NCHUNK, CHUNK)

    zeros_d = jnp.zeros((NPAD, D), jnp.float32)
    ones_d = jnp.ones((CHUNK, D), jnp.float32)

    degp = _deg_kernel()(dst_c, ones_d, zeros_d)
    h, z = _proj_ln(x, W_in, b_in.reshape(1, D),
                    ln_g[0].reshape(1, D), ln_b[0].reshape(1, D))

    out = None
    for i in range(W_self.shape[0]):
        parts = _edge_kernel()(idx_p, z, zeros_d)
        if i + 1 < W_self.shape[0]:
            h, z = _mid_layer(h, z, parts, degp, W_neigh[i], W_self[i],
                              b_conv[i].reshape(1, D),
                              ln_g[i + 1].reshape(1, D),
                              ln_b[i + 1].reshape(1, D))
        else:
            out = _last_layer(h, z, parts, degp, W_neigh[i], W_self[i],
                              b_conv[i].reshape(1, D), W_out,
                              b_out.reshape(1, DOUT))
    return out


# speedup vs baseline: 3.7815x; 1.2326x over previous
import functools

import jax
import jax.numpy as jnp
from jax import lax
from jax.experimental import pallas as pl
from jax.experimental.pallas import tpu as pltpu
from jax.experimental.pallas import tpu_sc as plsc

N = 10000
E = 320000
D = 128
DOUT = 64
NC = 2
NS = 16
NW = NC * NS
CHUNK = 128
NCHUNK = 80
EPAD = NW * NCHUNK * CHUNK
NPAD = 10112
RPT = NPAD // NS
BR = 2000


@functools.cache
def _mesh():
    return plsc.VectorSubcoreMesh(core_axis_name="c", subcore_axis_name="s",
                                  num_cores=NC, num_subcores=NS)


HC = NCHUNK // 2


def _edge_body(idx_hbm, z_hbm, zeros_hbm, out_hbm, idx_v, rows_v, acc,
               semg0, semg1):
    c = lax.axis_index("c")
    s = lax.axis_index("s")
    wid = s * NC + c
    pltpu.sync_copy(zeros_hbm.at[pl.ds(s * RPT, RPT)],
                    acc.at[pl.ds(s * RPT, RPT)])
    plsc.subcore_barrier()

    b0 = rows_v.at[0]
    b1 = rows_v.at[1]
    for h in range(2):
        pltpu.sync_copy(idx_hbm.at[wid, pl.ds(h * HC, HC)], idx_v)
        pltpu.async_copy(z_hbm.at[idx_v.at[0, 0]], b0, semg0)

        def pair(p, carry):
            j = 2 * p
            pltpu.make_async_copy(z_hbm.at[idx_v.at[j, 0]], b0, semg0).wait()
            pltpu.async_copy(z_hbm.at[idx_v.at[j + 1, 0]], b1, semg1)
            pltpu.sync_copy(b0, acc.at[idx_v.at[j, 1]], add=True)
            pltpu.make_async_copy(z_hbm.at[idx_v.at[j + 1, 0]], b1,
                                  semg1).wait()
            nxt = lax.rem(j + 2, HC)
            pltpu.async_copy(z_hbm.at[idx_v.at[nxt, 0]], b0, semg0)
            pltpu.sync_copy(b1, acc.at[idx_v.at[j + 1, 1]], add=True)
            return carry

        lax.fori_loop(0, HC // 2, pair, 0)
        pltpu.make_async_copy(z_hbm.at[idx_v.at[0, 0]], b0, semg0).wait()
    plsc.subcore_barrier()
    pltpu.sync_copy(acc.at[pl.ds(s * RPT, RPT)],
                    out_hbm.at[c, pl.ds(s * RPT, RPT)])


@functools.cache
def _edge_kernel():
    return pl.kernel(
        _edge_body,
        out_type=jax.ShapeDtypeStruct((NC, NPAD, D), jnp.float32),
        mesh=_mesh(),
        scratch_types=[
            pltpu.VMEM((HC, 2, CHUNK), jnp.int32),
            pltpu.VMEM((2, CHUNK, D), jnp.float32),
            pltpu.VMEM_SHARED((NPAD, D), jnp.float32),
            pltpu.SemaphoreType.DMA,
            pltpu.SemaphoreType.DMA,
        ],
    )


def _deg_body(dst_hbm, ones_hbm, zeros_hbm, out_hbm, dst_v, ones_v, acc):
    c = lax.axis_index("c")
    s = lax.axis_index("s")
    wid = s * NC + c
    pltpu.sync_copy(zeros_hbm.at[pl.ds(s * RPT, RPT)],
                    acc.at[pl.ds(s * RPT, RPT)])
    pltpu.sync_copy(ones_hbm, ones_v)
    pltpu.sync_copy(dst_hbm.at[wid], dst_v)
    plsc.subcore_barrier()

    def step(j, carry):
        pltpu.sync_copy(ones_v, acc.at[dst_v.at[j]], add=True)
        return carry

    lax.fori_loop(0, NCHUNK, step, 0)
    plsc.subcore_barrier()
    pltpu.sync_copy(acc.at[pl.ds(s * RPT, RPT)],
                    out_hbm.at[c, pl.ds(s * RPT, RPT)])


@functools.cache
def _deg_kernel():
    return pl.kernel(
        _deg_body,
        out_type=jax.ShapeDtypeStruct((NC, NPAD, D), jnp.float32),
        mesh=_mesh(),
        scratch_types=[
            pltpu.VMEM((NCHUNK, CHUNK), jnp.int32),
            pltpu.VMEM((CHUNK, D), jnp.float32),
            pltpu.VMEM_SHARED((NPAD, D), jnp.float32),
        ],
    )


def _ln(h, g, b):
    mu = jnp.mean(h, axis=-1, keepdims=True)
    var = jnp.mean((h - mu) ** 2, axis=-1, keepdims=True)
    return (h - mu) * lax.rsqrt(var + 1e-5) * g + b


def _proj_body(x_ref, w_ref, b_ref, g_ref, bb_ref, h_ref, z_ref):
    h = jnp.dot(x_ref[...], w_ref[...],
                preferred_element_type=jnp.float32) + b_ref[...]
    h_ref[...] = h
    z_ref[...] = _ln(h, g_ref[...], bb_ref[...])


_proj_ln = pl.pallas_call(
    _proj_body,
    grid=(N // BR,),
    in_specs=[
        pl.BlockSpec((BR, D), lambda i: (i, 0)),
        pl.BlockSpec((D, D), lambda i: (0, 0)),
        pl.BlockSpec((1, D), lambda i: (0, 0)),
        pl.BlockSpec((1, D), lambda i: (0, 0)),
        pl.BlockSpec((1, D), lambda i: (0, 0)),
    ],
    out_specs=[pl.BlockSpec((BR, D), lambda i: (i, 0)),
               pl.BlockSpec((BR, D), lambda i: (i, 0))],
    out_shape=[jax.ShapeDtypeStruct((N, D), jnp.float32),
               jax.ShapeDtypeStruct((N, D), jnp.float32)],
)


def _sage_common(h_ref, z_ref, p_ref, dg_ref, wn_ref, ws_ref, bc_ref):
    deg = jnp.maximum(dg_ref[0, :, 0:1] + dg_ref[1, :, 0:1], 1.0)
    agg = (p_ref[0] + p_ref[1]) / deg
    conv = (jnp.dot(agg, wn_ref[...], preferred_element_type=jnp.float32)
            + jnp.dot(z_ref[...], ws_ref[...], preferred_element_type=jnp.float32)
            + bc_ref[...])
    return jnp.maximum(h_ref[...] + conv, 0.0)


def _mid_body(h_ref, z_ref, p_ref, dg_ref, wn_ref, ws_ref, bc_ref,
              g_ref, bb_ref, ho_ref, zo_ref):
    hn = _sage_common(h_ref, z_ref, p_ref, dg_ref, wn_ref, ws_ref, bc_ref)
    ho_ref[...] = hn
    zo_ref[...] = _ln(hn, g_ref[...], bb_ref[...])


_SAGE_SPECS = [
    pl.BlockSpec((BR, D), lambda i: (i, 0)),
    pl.BlockSpec((BR, D), lambda i: (i, 0)),
    pl.BlockSpec((NC, BR, D), lambda i: (0, i, 0)),
    pl.BlockSpec((NC, BR, D), lambda i: (0, i, 0)),
    pl.BlockSpec((D, D), lambda i: (0, 0)),
    pl.BlockSpec((D, D), lambda i: (0, 0)),
    pl.BlockSpec((1, D), lambda i: (0, 0)),
]

_mid_layer = pl.pallas_call(
    _mid_body,
    grid=(N // BR,),
    in_specs=_SAGE_SPECS + [
        pl.BlockSpec((1, D), lambda i: (0, 0)),
        pl.BlockSpec((1, D), lambda i: (0, 0)),
    ],
    out_specs=[pl.BlockSpec((BR, D), lambda i: (i, 0)),
               pl.BlockSpec((BR, D), lambda i: (i, 0))],
    out_shape=[jax.ShapeDtypeStruct((N, D), jnp.float32),
               jax.ShapeDtypeStruct((N, D), jnp.float32)],
)


def _last_body(h_ref, z_ref, p_ref, dg_ref, wn_ref, ws_ref, bc_ref,
               wo_ref, bo_ref, o_ref):
    hn = _sage_common(h_ref, z_ref, p_ref, dg_ref, wn_ref, ws_ref, bc_ref)
    o_ref[...] = jnp.dot(hn, wo_ref[...],
                         preferred_element_type=jnp.float32) + bo_ref[...]


_last_layer = pl.pallas_call(
    _last_body,
    grid=(N // BR,),
    in_specs=_SAGE_SPECS + [
        pl.BlockSpec((D, DOUT), lambda i: (0, 0)),
        pl.BlockSpec((1, DOUT), lambda i: (0, 0)),
    ],
    out_specs=pl.BlockSpec((BR, DOUT), lambda i: (i, 0)),
    out_shape=jax.ShapeDtypeStruct((N, DOUT), jnp.float32),
)


def kernel(x, edge_index, W_in, b_in, ln_g, ln_b, W_neigh, W_self, b_conv,
           W_out, b_out):
    i32 = jnp.int32
    src = edge_index[0].astype(i32)
    dst = edge_index[1].astype(i32)
    pad = jnp.arange(EPAD - E, dtype=i32)
    src_p = jnp.concatenate([src, pad % N])
    dst_p = jnp.concatenate([dst, N + pad % (NPAD - N)])
    idx_p = jnp.stack([src_p.reshape(NW, NCHUNK, CHUNK),
                       dst_p.reshape(NW, NCHUNK, CHUNK)], axis=2)
    dst_c = dst_p.reshape(NW, NCHUNK, CHUNK)

    zeros_d = jnp.zeros((NPAD, D), jnp.float32)
    ones_d = jnp.ones((CHUNK, D), jnp.float32)

    degp = _deg_kernel()(dst_c, ones_d, zeros_d)
    h, z = _proj_ln(x, W_in, b_in.reshape(1, D),
                    ln_g[0].reshape(1, D), ln_b[0].reshape(1, D))

    out = None
    for i in range(W_self.shape[0]):
        parts = _edge_kernel()(idx_p, z, zeros_d)
        if i + 1 < W_self.shape[0]:
            h, z = _mid_layer(h, z, parts, degp, W_neigh[i], W_self[i],
                              b_conv[i].reshape(1, D),
                              ln_g[i + 1].reshape(1, D),
                              ln_b[i + 1].reshape(1, D))
        else:
            out = _last_layer(h, z, parts, degp, W_neigh[i], W_self[i],
                              b_conv[i].reshape(1, D), W_out,
                              b_out.reshape(1, DOUT))
    return out
